# baseline (speedup 1.0000x reference)
import functools

import jax
import jax.numpy as jnp
from jax import lax
from jax.experimental import pallas as pl
from jax.experimental.pallas import tpu as pltpu
from jax.experimental.pallas import tpu_sc as plsc

B = 4096
L = 50
LRA_IN = 240
H = 256
LRA_EMB = 128
NODE_DIM = 64
OUT_DIM = LRA_EMB + NODE_DIM
BB = 512
PAIR = BB // 2


@functools.lru_cache(maxsize=None)
def _make_sc_gather():
    nc, ns = 2, 16
    nw = nc * ns
    bpw = B // nw
    mesh = plsc.VectorSubcoreMesh(core_axis_name="c", subcore_axis_name="s",
                                  num_cores=nc)

    @functools.partial(
        pl.kernel,
        mesh=mesh,
        compiler_params=pltpu.CompilerParams(use_tc_tiling_on_sc=False),
        out_type=jax.ShapeDtypeStruct((L, B // 2, 128), jnp.float32),
        scratch_types=[
            pltpu.VMEM((bpw,), jnp.int32),
            pltpu.VMEM((bpw, NODE_DIM), jnp.float32),
            pltpu.SemaphoreType.DMA,
        ],
    )
    def sc_gather(table_hbm, idst_hbm, out_hbm, idx_v, buf, sem):
        wid = lax.axis_index("s") * nc + lax.axis_index("c")
        b0 = wid * bpw
        blk = b0 // BB
        within = b0 % BB
        col = jnp.where(within < PAIR, 0, NODE_DIM)
        row_c = blk * PAIR + within % PAIR

        def body(l, _):
            pltpu.sync_copy(idst_hbm.at[l, pl.ds(b0, bpw)], idx_v)
            pltpu.async_copy(table_hbm.at[idx_v], buf, sem).wait()
            pltpu.sync_copy(
                buf, out_hbm.at[l, pl.ds(row_c, bpw), pl.ds(col, NODE_DIM)])
            return ()

        lax.fori_loop(0, L, body, (), unroll=False)

    return sc_gather


def _mlp_body(x_ref, w1t_ref, b1_ref, w2t_ref, b2_ref, node_ref, o_ref):
    x = x_ref[0].astype(jnp.bfloat16)
    h = jnp.dot(w1t_ref[...], x, preferred_element_type=jnp.float32)
    h = h + b1_ref[...]
    h = h * jax.nn.sigmoid(h)
    y = jnp.dot(w2t_ref[...], h.astype(jnp.bfloat16),
                preferred_element_type=jnp.float32)
    y = y + b2_ref[...]
    t = jnp.transpose(node_ref[0], (1, 0))
    node = jnp.concatenate([t[0:NODE_DIM, :], t[NODE_DIM:, :]], axis=1)
    o_ref[0] = jnp.concatenate([y, node], axis=0)


def _mlp_concat(xt, w1t, b1c, w2t, b2c, node3d):
    return pl.pallas_call(
        _mlp_body,
        grid=(L, B // BB),
        in_specs=[
            pl.BlockSpec((1, LRA_IN, BB), lambda l, j: (l, 0, j)),
            pl.BlockSpec((H, LRA_IN), lambda l, j: (0, 0)),
            pl.BlockSpec((H, 1), lambda l, j: (0, 0)),
            pl.BlockSpec((LRA_EMB, H), lambda l, j: (0, 0)),
            pl.BlockSpec((LRA_EMB, 1), lambda l, j: (0, 0)),
            pl.BlockSpec((1, PAIR, 128), lambda l, j: (l, j, 0)),
        ],
        out_specs=pl.BlockSpec((1, OUT_DIM, BB), lambda l, j: (l, 0, j)),
        out_shape=jax.ShapeDtypeStruct((L, OUT_DIM, B), jnp.float32),
    )(xt, w1t, b1c, w2t, b2c, node3d)


def kernel(precomputed_lra_batch, nearest_node_ids, W1, b1, W2, b2,
           road_node_embeddings):
    xt = jnp.transpose(precomputed_lra_batch, (1, 2, 0))
    idst = jnp.transpose(nearest_node_ids, (1, 0)).astype(jnp.int32)
    node3d = _make_sc_gather()(road_node_embeddings, idst)
    outt = _mlp_concat(xt, W1.T.astype(jnp.bfloat16), b1.reshape(H, 1),
                       W2.T.astype(jnp.bfloat16),
                       b2.reshape(LRA_EMB, 1), node3d)
    return jnp.transpose(outt, (2, 0, 1))


# speedup vs baseline: 1.6737x; 1.6737x over previous
import functools

import jax
import jax.numpy as jnp
from jax import lax
from jax.experimental import pallas as pl
from jax.experimental.pallas import tpu as pltpu
from jax.experimental.pallas import tpu_sc as plsc

B = 4096
L = 50
LRA_IN = 240
H = 256
LRA_EMB = 128
NODE_DIM = 64
OUT_DIM = LRA_EMB + NODE_DIM
BB = 4096
PAIR = BB // 2


@functools.lru_cache(maxsize=None)
def _make_sc_gather():
    nc, ns = 2, 16
    nw = nc * ns
    bpw = B // nw
    mesh = plsc.VectorSubcoreMesh(core_axis_name="c", subcore_axis_name="s",
                                  num_cores=nc)

    @functools.partial(
        pl.kernel,
        mesh=mesh,
        compiler_params=pltpu.CompilerParams(use_tc_tiling_on_sc=False),
        out_type=jax.ShapeDtypeStruct((L, B // 2, 128), jnp.float32),
        scratch_types=[
            pltpu.VMEM((bpw,), jnp.int32),
            pltpu.VMEM((bpw, NODE_DIM), jnp.float32),
            pltpu.SemaphoreType.DMA,
        ],
    )
    def sc_gather(table_hbm, idst_hbm, out_hbm, idx_v, buf, sem):
        wid = lax.axis_index("s") * nc + lax.axis_index("c")
        b0 = wid * bpw
        blk = b0 // BB
        within = b0 % BB
        col = jnp.where(within < PAIR, 0, NODE_DIM)
        row_c = blk * PAIR + within % PAIR

        def body(l, _):
            pltpu.sync_copy(idst_hbm.at[l, pl.ds(b0, bpw)], idx_v)
            pltpu.async_copy(table_hbm.at[idx_v], buf, sem).wait()
            pltpu.sync_copy(
                buf, out_hbm.at[l, pl.ds(row_c, bpw), pl.ds(col, NODE_DIM)])
            return ()

        lax.fori_loop(0, L, body, (), unroll=False)

    return sc_gather


def _mlp_body(x_ref, w1t_ref, b1_ref, w2t_ref, b2_ref, node_ref, o_ref):
    x = x_ref[0].astype(jnp.bfloat16)
    h = jnp.dot(w1t_ref[...], x, preferred_element_type=jnp.float32)
    h = h + b1_ref[...]
    h = h * jax.nn.sigmoid(h)
    y = jnp.dot(w2t_ref[...], h.astype(jnp.bfloat16),
                preferred_element_type=jnp.float32)
    y = y + b2_ref[...]
    t = jnp.transpose(node_ref[0], (1, 0))
    node = jnp.concatenate([t[0:NODE_DIM, :], t[NODE_DIM:, :]], axis=1)
    o_ref[0] = jnp.concatenate([y, node], axis=0)


def _mlp_concat(xt, w1t, b1c, w2t, b2c, node3d):
    return pl.pallas_call(
        _mlp_body,
        grid=(L, B // BB),
        in_specs=[
            pl.BlockSpec((1, LRA_IN, BB), lambda l, j: (l, 0, j)),
            pl.BlockSpec((H, LRA_IN), lambda l, j: (0, 0)),
            pl.BlockSpec((H, 1), lambda l, j: (0, 0)),
            pl.BlockSpec((LRA_EMB, H), lambda l, j: (0, 0)),
            pl.BlockSpec((LRA_EMB, 1), lambda l, j: (0, 0)),
            pl.BlockSpec((1, PAIR, 128), lambda l, j: (l, j, 0)),
        ],
        out_specs=pl.BlockSpec((1, OUT_DIM, BB), lambda l, j: (l, 0, j)),
        out_shape=jax.ShapeDtypeStruct((L, OUT_DIM, B), jnp.float32),
    )(xt, w1t, b1c, w2t, b2c, node3d)


def kernel(precomputed_lra_batch, nearest_node_ids, W1, b1, W2, b2,
           road_node_embeddings):
    xt = jnp.transpose(precomputed_lra_batch, (1, 2, 0))
    idst = jnp.transpose(nearest_node_ids, (1, 0)).astype(jnp.int32)
    node3d = _make_sc_gather()(road_node_embeddings, idst)
    outt = _mlp_concat(xt, W1.T.astype(jnp.bfloat16), b1.reshape(H, 1),
                       W2.T.astype(jnp.bfloat16),
                       b2.reshape(LRA_EMB, 1), node3d)
    return jnp.transpose(outt, (2, 0, 1))


# speedup vs baseline: 1.9404x; 1.1593x over previous
import functools

import jax
import jax.numpy as jnp
from jax import lax
from jax.experimental import pallas as pl
from jax.experimental.pallas import tpu as pltpu
from jax.experimental.pallas import tpu_sc as plsc

B = 4096
L = 50
LRA_IN = 240
H = 256
LRA_EMB = 128
NODE_DIM = 64
OUT_DIM = LRA_EMB + NODE_DIM
BB = 4096
PAIR = BB // 2


@functools.lru_cache(maxsize=None)
def _make_sc_gather():
    nc, ns = 2, 16
    nw = nc * ns
    bpw = B // nw
    mesh = plsc.VectorSubcoreMesh(core_axis_name="c", subcore_axis_name="s",
                                  num_cores=nc)

    @functools.partial(
        pl.kernel,
        mesh=mesh,
        compiler_params=pltpu.CompilerParams(use_tc_tiling_on_sc=False),
        out_type=jax.ShapeDtypeStruct((L, B // 2, 128), jnp.float32),
        scratch_types=[
            pltpu.VMEM((bpw,), jnp.int32),
            pltpu.VMEM((bpw, NODE_DIM), jnp.float32),
            pltpu.SemaphoreType.DMA,
        ],
    )
    def sc_gather(table_hbm, idst_hbm, out_hbm, idx_v, buf, sem):
        wid = lax.axis_index("s") * nc + lax.axis_index("c")
        b0 = wid * bpw
        blk = b0 // BB
        within = b0 % BB
        col = jnp.where(within < PAIR, 0, NODE_DIM)
        row_c = blk * PAIR + within % PAIR

        def body(l, _):
            pltpu.sync_copy(idst_hbm.at[l, pl.ds(b0, bpw)], idx_v)
            pltpu.async_copy(table_hbm.at[idx_v], buf, sem).wait()
            pltpu.sync_copy(
                buf, out_hbm.at[l, pl.ds(row_c, bpw), pl.ds(col, NODE_DIM)])
            return ()

        lax.fori_loop(0, L, body, (), unroll=False)

    return sc_gather


def _mlp_body(x_ref, w1t_ref, b1_ref, w2t_ref, b2_ref, o_ref):
    x = x_ref[0].astype(jnp.bfloat16)
    h = jnp.dot(w1t_ref[...], x, preferred_element_type=jnp.float32)
    h = h + b1_ref[...]
    h = h * jax.nn.sigmoid(h)
    y = jnp.dot(w2t_ref[...], h.astype(jnp.bfloat16),
                preferred_element_type=jnp.float32)
    o_ref[0] = y + b2_ref[...]


def _mlp_pass(xt, w1t, b1c, w2t, b2c):
    return pl.pallas_call(
        _mlp_body,
        grid=(L, B // BB),
        in_specs=[
            pl.BlockSpec((1, LRA_IN, BB), lambda l, j: (l, 0, j)),
            pl.BlockSpec((H, LRA_IN), lambda l, j: (0, 0)),
            pl.BlockSpec((H, 1), lambda l, j: (0, 0)),
            pl.BlockSpec((LRA_EMB, H), lambda l, j: (0, 0)),
            pl.BlockSpec((LRA_EMB, 1), lambda l, j: (0, 0)),
        ],
        out_specs=pl.BlockSpec((1, LRA_EMB, BB), lambda l, j: (l, 0, j)),
        out_shape=jax.ShapeDtypeStruct((L, OUT_DIM, B), jnp.float32),
    )(xt, w1t, b1c, w2t, b2c)


def _node_body(node_ref, alias_ref, o_ref):
    del alias_ref
    t = jnp.transpose(node_ref[0], (1, 0))
    o_ref[0] = jnp.concatenate([t[0:NODE_DIM, :], t[NODE_DIM:, :]], axis=1)


def _node_pass(node3d, outt):
    return pl.pallas_call(
        _node_body,
        grid=(L, B // BB),
        in_specs=[
            pl.BlockSpec((1, PAIR, 128), lambda l, j: (l, j, 0)),
            pl.BlockSpec(memory_space=pl.MemorySpace.ANY),
        ],
        out_specs=pl.BlockSpec((1, NODE_DIM, BB), lambda l, j: (l, 2, j)),
        out_shape=jax.ShapeDtypeStruct((L, OUT_DIM, B), jnp.float32),
        input_output_aliases={1: 0},
    )(node3d, outt)


def kernel(precomputed_lra_batch, nearest_node_ids, W1, b1, W2, b2,
           road_node_embeddings):
    xt = jnp.transpose(precomputed_lra_batch, (1, 2, 0))
    idst = jnp.transpose(nearest_node_ids, (1, 0)).astype(jnp.int32)
    node3d = _make_sc_gather()(road_node_embeddings, idst)
    outt = _mlp_pass(xt, W1.T.astype(jnp.bfloat16), b1.reshape(H, 1),
                     W2.T.astype(jnp.bfloat16), b2.reshape(LRA_EMB, 1))
    outt = _node_pass(node3d, outt)
    return jnp.transpose(outt, (2, 0, 1))
